# baseline (speedup 1.0000x reference)
import functools

import jax
import jax.numpy as jnp
from jax import lax
from jax.experimental import pallas as pl
from jax.experimental.pallas import tpu as pltpu
from jax.experimental.pallas import tpu_sc as plsc

B = 4096
N = 1000
NPAD = 1024
TOPK = 5
TCOST = 0.001
A_CONST = 249500250.0
DENOM = 83333250.0 + 1e-8

ROWS_BLK = 64
NSTEPS = B // ROWS_BLK


_NSTAGE = 55


def _pack_key(x, col):
    bits = lax.bitcast_convert_type(x, jnp.int32)
    mono = jnp.where(bits < 0, bits ^ jnp.int32(0x7FFFFFFF), bits)
    return (mono & jnp.int32(-1024)) | col


def _sort_body(p_ref, t_ref, sp_ref, st_ref, mse_ref):
    pid = pl.program_id(0)
    p = p_ref[...]
    t = t_ref[...]
    col = lax.broadcasted_iota(jnp.int32, (ROWS_BLK, NPAD), 1)
    valid = col < N
    diff = jnp.where(valid, p - t, 0.0)
    msep = jnp.sum(diff * diff)

    @pl.when(pid == 0)
    def _():
        mse_ref[...] = jnp.zeros_like(mse_ref)

    mse_ref[...] += jnp.reshape(msep, (1, 1))

    def substage(key, k, d, bit0, take_min):
        up = pltpu.roll(key, NPAD - d, 1)
        dn = pltpu.roll(key, d, 1)
        pk = jnp.where(bit0, up, dn)
        lt = key < pk
        return jnp.where(lt == take_min, key, pk)

    kp = _pack_key(p, col)
    kt = _pack_key(t, col)
    k = 2
    while k <= NPAD:
        d = k // 2
        while d >= 1:
            bit0 = (col & d) == 0
            take_min = ((col & k) == 0) == bit0
            kp = substage(kp, k, d, bit0, take_min)
            kt = substage(kt, k, d, bit0, take_min)
            d //= 2
        k *= 2

    sp_ref[...] = kp & jnp.int32(NPAD - 1)
    st_ref[...] = kt & jnp.int32(NPAD - 1)


def _run_sort(pred_pad, targ_pad):
    return pl.pallas_call(
        _sort_body,
        grid=(NSTEPS,),
        in_specs=[
            pl.BlockSpec((ROWS_BLK, NPAD), lambda i: (i, 0)),
            pl.BlockSpec((ROWS_BLK, NPAD), lambda i: (i, 0)),
        ],
        out_specs=[
            pl.BlockSpec((ROWS_BLK, NPAD), lambda i: (i, 0)),
            pl.BlockSpec((ROWS_BLK, NPAD), lambda i: (i, 0)),
            pl.BlockSpec((1, 1), lambda i: (0, 0)),
        ],
        out_shape=[
            jax.ShapeDtypeStruct((B, NPAD), jnp.int32),
            jax.ShapeDtypeStruct((B, NPAD), jnp.int32),
            jax.ShapeDtypeStruct((1, 1), jnp.float32),
        ],
    )(pred_pad, targ_pad)


NC = 2
NS = 16
NW = NC * NS
RPW = B // NW
CHUNK = 16
NCHUNK = RPW // CHUNK

_VREGS = NPAD // 16


def _sc_row_compute(i, sp_v, st_v, tv_v, tr_v, ic_buf, ret_buf):
    sbase = i * NPAD
    tbase = i * N
    lanes = lax.iota(jnp.int32, 16)
    for k in range(_VREGS):
        idx = st_v[pl.ds(sbase + k * 16, 16)]
        val = (lanes + (k * 16)).astype(jnp.float32)
        plsc.store_scatter(tr_v, [idx], val)
    acc = jnp.zeros((16,), jnp.float32)
    for k in range(63):
        ip = sp_v[pl.ds(sbase + k * 16, 16)]
        g = plsc.load_gather(tr_v, [ip])
        p_idx = lanes + (k * 16)
        w = jnp.where(p_idx < N, p_idx, 0).astype(jnp.float32)
        acc = acc + w * g
    icn = jnp.sum(acc)
    ib = sp_v[pl.ds(sbase, 16)]
    gb = plsc.load_gather(tv_v, [jnp.minimum(ib, N - 1) + tbase])
    bsum = jnp.sum(jnp.where(lanes < TOPK, gb, 0.0))
    it = sp_v[pl.ds(sbase + 992, 16)]
    gt = plsc.load_gather(tv_v, [jnp.minimum(it, N - 1) + tbase])
    tmask = (lanes >= 3) & (lanes < 8)
    tsum = jnp.sum(jnp.where(tmask, gt, 0.0))
    ret = (1.0 / TOPK) * (tsum - bsum) - TCOST * 2.0

    lane0 = lanes == 0
    widx = jnp.full((16,), i, jnp.int32)
    plsc.store_scatter(ic_buf, [widx], jnp.full((16,), icn, jnp.float32),
                       mask=lane0)
    plsc.store_scatter(ret_buf, [widx], jnp.full((16,), ret, jnp.float32),
                       mask=lane0)


def _sc_body(sp_hbm, st_hbm, tgt_hbm, icnum_hbm, ret_hbm,
             sp_v, st_v, tv_v, tr_v, ic_buf, ret_buf):
    wid = lax.axis_index("s") * NC + lax.axis_index("c")
    base = wid * RPW

    def chunk_fn(c, carry):
        r0 = base + c * CHUNK
        pltpu.sync_copy(sp_hbm.at[pl.ds(r0 * NPAD, CHUNK * NPAD)], sp_v)
        pltpu.sync_copy(st_hbm.at[pl.ds(r0 * NPAD, CHUNK * NPAD)], st_v)
        pltpu.sync_copy(tgt_hbm.at[pl.ds(r0 * N, CHUNK * N)], tv_v)

        def row_fn(i, carry2):
            _sc_row_compute(i, sp_v, st_v, tv_v, tr_v,
                            ic_buf.at[pl.ds(c * CHUNK, CHUNK)],
                            ret_buf.at[pl.ds(c * CHUNK, CHUNK)])
            return carry2

        lax.fori_loop(0, CHUNK, row_fn, 0)
        return carry

    lax.fori_loop(0, NCHUNK, chunk_fn, 0)
    pltpu.sync_copy(ic_buf, icnum_hbm.at[pl.ds(base, RPW)])
    pltpu.sync_copy(ret_buf, ret_hbm.at[pl.ds(base, RPW)])


def _run_sc(sp_flat, st_flat, tgt_flat):
    mesh = plsc.VectorSubcoreMesh(core_axis_name="c", subcore_axis_name="s")
    fn = functools.partial(
        pl.kernel,
        mesh=mesh,
        compiler_params=pltpu.CompilerParams(needs_layout_passes=False),
        out_type=[
            jax.ShapeDtypeStruct((B,), jnp.float32),
            jax.ShapeDtypeStruct((B,), jnp.float32),
        ],
        scratch_types=[
            pltpu.VMEM((CHUNK * NPAD,), jnp.int32),
            pltpu.VMEM((CHUNK * NPAD,), jnp.int32),
            pltpu.VMEM((CHUNK * N,), jnp.float32),
            pltpu.VMEM((NPAD,), jnp.float32),
            pltpu.VMEM((RPW,), jnp.float32),
            pltpu.VMEM((RPW,), jnp.float32),
        ],
    )(_sc_body)
    return fn(sp_flat, st_flat, tgt_flat)


def _combine_body(ic_ref, ret_ref, mse_ref, fw_ref, out_ref):
    icn = ic_ref[...]
    ret = ret_ref[...]
    nb = float(B)
    ic_loss = -(jnp.sum(icn) - nb * A_CONST) / (DENOM * nb)
    mean_ret = jnp.sum(ret) / nb
    var = jnp.sum((ret - mean_ret) ** 2) / (nb - 1.0)
    sharpe_loss = -(mean_ret / (jnp.sqrt(var) + 1e-8))
    l1 = jnp.sum(jnp.abs(fw_ref[...]))
    mse = mse_ref[0, 0] / (nb * float(N))
    total = mse + 0.5 * ic_loss + 0.5 * sharpe_loss + 0.0001 * l1
    out_ref[...] = jnp.reshape(total, (1, 1))


def _run_combine(icnum, ret, msesum, factor_weights):
    return pl.pallas_call(
        _combine_body,
        out_shape=jax.ShapeDtypeStruct((1, 1), jnp.float32),
    )(icnum.reshape(1, B), ret.reshape(1, B), msesum, factor_weights)


def kernel(predictions, targets, factor_weights):
    pred_pad = jnp.pad(predictions, ((0, 0), (0, NPAD - N)),
                       constant_values=jnp.inf)
    targ_pad = jnp.pad(targets, ((0, 0), (0, NPAD - N)),
                       constant_values=jnp.inf)
    sp, st, msesum = _run_sort(pred_pad, targ_pad)
    icnum, ret = _run_sc(sp.reshape(-1), st.reshape(-1), targets.reshape(-1))
    out = _run_combine(icnum, ret, msesum, factor_weights)
    return out.reshape(())


# speedup vs baseline: 1.0039x; 1.0039x over previous
import functools

import jax
import jax.numpy as jnp
from jax import lax
from jax.experimental import pallas as pl
from jax.experimental.pallas import tpu as pltpu
from jax.experimental.pallas import tpu_sc as plsc

B = 4096
N = 1000
NPAD = 1024
TOPK = 5
TCOST = 0.001
A_CONST = 249500250.0
DENOM = 83333250.0 + 1e-8

ROWS_BLK = 32
NSTEPS = B // ROWS_BLK


_NSTAGE = 55


def _pack_key(x, col):
    bits = lax.bitcast_convert_type(x, jnp.int32)
    mono = jnp.where(bits < 0, bits ^ jnp.int32(0x7FFFFFFF), bits)
    return (mono & jnp.int32(-1024)) | col


def _sort_body(p_ref, t_ref, sp_ref, st_ref, mse_ref):
    pid = pl.program_id(0)
    p = p_ref[...]
    t = t_ref[...]
    col = lax.broadcasted_iota(jnp.int32, (ROWS_BLK, NPAD), 1)
    valid = col < N
    diff = jnp.where(valid, p - t, 0.0)
    msep = jnp.sum(diff * diff)

    @pl.when(pid == 0)
    def _():
        mse_ref[...] = jnp.zeros_like(mse_ref)

    mse_ref[...] += jnp.reshape(msep, (1, 1))

    def substage(key, k, d, bit0, take_min):
        up = pltpu.roll(key, NPAD - d, 1)
        dn = pltpu.roll(key, d, 1)
        pk = jnp.where(bit0, up, dn)
        lt = key < pk
        return jnp.where(lt == take_min, key, pk)

    kp = _pack_key(p, col)
    kt = _pack_key(t, col)
    k = 2
    while k <= NPAD:
        d = k // 2
        while d >= 1:
            bit0 = (col & d) == 0
            take_min = ((col & k) == 0) == bit0
            kp = substage(kp, k, d, bit0, take_min)
            kt = substage(kt, k, d, bit0, take_min)
            d //= 2
        k *= 2

    sp_ref[...] = kp & jnp.int32(NPAD - 1)
    st_ref[...] = kt & jnp.int32(NPAD - 1)


def _run_sort(pred_pad, targ_pad):
    return pl.pallas_call(
        _sort_body,
        grid=(NSTEPS,),
        in_specs=[
            pl.BlockSpec((ROWS_BLK, NPAD), lambda i: (i, 0)),
            pl.BlockSpec((ROWS_BLK, NPAD), lambda i: (i, 0)),
        ],
        out_specs=[
            pl.BlockSpec((ROWS_BLK, NPAD), lambda i: (i, 0)),
            pl.BlockSpec((ROWS_BLK, NPAD), lambda i: (i, 0)),
            pl.BlockSpec((1, 1), lambda i: (0, 0)),
        ],
        out_shape=[
            jax.ShapeDtypeStruct((B, NPAD), jnp.int32),
            jax.ShapeDtypeStruct((B, NPAD), jnp.int32),
            jax.ShapeDtypeStruct((1, 1), jnp.float32),
        ],
    )(pred_pad, targ_pad)


NC = 2
NS = 16
NW = NC * NS
RPW = B // NW
CHUNK = 16
NCHUNK = RPW // CHUNK

_VREGS = NPAD // 16


def _sc_row_compute(i, sp_v, st_v, tv_v, tr_v, ic_buf, ret_buf):
    sbase = i * NPAD
    tbase = i * N
    lanes = lax.iota(jnp.int32, 16)
    for k in range(_VREGS):
        idx = st_v[pl.ds(sbase + k * 16, 16)]
        val = (lanes + (k * 16)).astype(jnp.float32)
        plsc.store_scatter(tr_v, [idx], val)
    acc = jnp.zeros((16,), jnp.float32)
    for k in range(63):
        ip = sp_v[pl.ds(sbase + k * 16, 16)]
        g = plsc.load_gather(tr_v, [ip])
        p_idx = lanes + (k * 16)
        w = jnp.where(p_idx < N, p_idx, 0).astype(jnp.float32)
        acc = acc + w * g
    icn = jnp.sum(acc)
    ib = sp_v[pl.ds(sbase, 16)]
    gb = plsc.load_gather(tv_v, [jnp.minimum(ib, N - 1) + tbase])
    bsum = jnp.sum(jnp.where(lanes < TOPK, gb, 0.0))
    it = sp_v[pl.ds(sbase + 992, 16)]
    gt = plsc.load_gather(tv_v, [jnp.minimum(it, N - 1) + tbase])
    tmask = (lanes >= 3) & (lanes < 8)
    tsum = jnp.sum(jnp.where(tmask, gt, 0.0))
    ret = (1.0 / TOPK) * (tsum - bsum) - TCOST * 2.0

    lane0 = lanes == 0
    widx = jnp.full((16,), i, jnp.int32)
    plsc.store_scatter(ic_buf, [widx], jnp.full((16,), icn, jnp.float32),
                       mask=lane0)
    plsc.store_scatter(ret_buf, [widx], jnp.full((16,), ret, jnp.float32),
                       mask=lane0)


def _sc_body(sp_hbm, st_hbm, tgt_hbm, icnum_hbm, ret_hbm,
             sp_v, st_v, tv_v, tr_v, ic_buf, ret_buf):
    wid = lax.axis_index("s") * NC + lax.axis_index("c")
    base = wid * RPW

    def chunk_fn(c, carry):
        r0 = base + c * CHUNK
        pltpu.sync_copy(sp_hbm.at[pl.ds(r0 * NPAD, CHUNK * NPAD)], sp_v)
        pltpu.sync_copy(st_hbm.at[pl.ds(r0 * NPAD, CHUNK * NPAD)], st_v)
        pltpu.sync_copy(tgt_hbm.at[pl.ds(r0 * N, CHUNK * N)], tv_v)

        def row_fn(i, carry2):
            _sc_row_compute(i, sp_v, st_v, tv_v, tr_v,
                            ic_buf.at[pl.ds(c * CHUNK, CHUNK)],
                            ret_buf.at[pl.ds(c * CHUNK, CHUNK)])
            return carry2

        lax.fori_loop(0, CHUNK, row_fn, 0)
        return carry

    lax.fori_loop(0, NCHUNK, chunk_fn, 0)
    pltpu.sync_copy(ic_buf, icnum_hbm.at[pl.ds(base, RPW)])
    pltpu.sync_copy(ret_buf, ret_hbm.at[pl.ds(base, RPW)])


def _run_sc(sp_flat, st_flat, tgt_flat):
    mesh = plsc.VectorSubcoreMesh(core_axis_name="c", subcore_axis_name="s")
    fn = functools.partial(
        pl.kernel,
        mesh=mesh,
        compiler_params=pltpu.CompilerParams(needs_layout_passes=False),
        out_type=[
            jax.ShapeDtypeStruct((B,), jnp.float32),
            jax.ShapeDtypeStruct((B,), jnp.float32),
        ],
        scratch_types=[
            pltpu.VMEM((CHUNK * NPAD,), jnp.int32),
            pltpu.VMEM((CHUNK * NPAD,), jnp.int32),
            pltpu.VMEM((CHUNK * N,), jnp.float32),
            pltpu.VMEM((NPAD,), jnp.float32),
            pltpu.VMEM((RPW,), jnp.float32),
            pltpu.VMEM((RPW,), jnp.float32),
        ],
    )(_sc_body)
    return fn(sp_flat, st_flat, tgt_flat)


def _combine_body(ic_ref, ret_ref, mse_ref, fw_ref, out_ref):
    icn = ic_ref[...]
    ret = ret_ref[...]
    nb = float(B)
    ic_loss = -(jnp.sum(icn) - nb * A_CONST) / (DENOM * nb)
    mean_ret = jnp.sum(ret) / nb
    var = jnp.sum((ret - mean_ret) ** 2) / (nb - 1.0)
    sharpe_loss = -(mean_ret / (jnp.sqrt(var) + 1e-8))
    l1 = jnp.sum(jnp.abs(fw_ref[...]))
    mse = mse_ref[0, 0] / (nb * float(N))
    total = mse + 0.5 * ic_loss + 0.5 * sharpe_loss + 0.0001 * l1
    out_ref[...] = jnp.reshape(total, (1, 1))


def _run_combine(icnum, ret, msesum, factor_weights):
    return pl.pallas_call(
        _combine_body,
        out_shape=jax.ShapeDtypeStruct((1, 1), jnp.float32),
    )(icnum.reshape(1, B), ret.reshape(1, B), msesum, factor_weights)


def kernel(predictions, targets, factor_weights):
    pred_pad = jnp.pad(predictions, ((0, 0), (0, NPAD - N)),
                       constant_values=jnp.inf)
    targ_pad = jnp.pad(targets, ((0, 0), (0, NPAD - N)),
                       constant_values=jnp.inf)
    sp, st, msesum = _run_sort(pred_pad, targ_pad)
    icnum, ret = _run_sc(sp.reshape(-1), st.reshape(-1), targets.reshape(-1))
    out = _run_combine(icnum, ret, msesum, factor_weights)
    return out.reshape(())


# speedup vs baseline: 1.1423x; 1.1378x over previous
import functools

import jax
import jax.numpy as jnp
from jax import lax
from jax.experimental import pallas as pl
from jax.experimental.pallas import tpu as pltpu
from jax.experimental.pallas import tpu_sc as plsc

B = 4096
N = 1000
NPAD = 1024
TOPK = 5
TCOST = 0.001
A_CONST = 249500250.0
DENOM = 83333250.0 + 1e-8

ROWS_BLK = 32
NSTEPS = B // ROWS_BLK


_NSTAGE = 55


def _pack_key(x, col):
    bits = lax.bitcast_convert_type(x, jnp.int32)
    mono = jnp.where(bits < 0, bits ^ jnp.int32(0x7FFFFFFF), bits)
    return (mono & jnp.int32(-1024)) | col


def _sort_body(p_ref, t_ref, sp_ref, mse_ref):
    pid = pl.program_id(0)
    p = p_ref[...]
    t = t_ref[...]
    col = lax.broadcasted_iota(jnp.int32, (ROWS_BLK, NPAD), 1)
    valid = col < N
    diff = jnp.where(valid, p - t, 0.0)
    msep = jnp.sum(diff * diff)

    @pl.when(pid == 0)
    def _():
        mse_ref[...] = jnp.zeros_like(mse_ref)

    mse_ref[...] += jnp.reshape(msep, (1, 1))

    def substage(key, k, d, bit0, take_min):
        up = pltpu.roll(key, NPAD - d, 1)
        dn = pltpu.roll(key, d, 1)
        pk = jnp.where(bit0, up, dn)
        mn = jnp.minimum(key, pk)
        mx = jnp.maximum(key, pk)
        return jnp.where(take_min, mn, mx)

    kp = _pack_key(p, col)
    kt = _pack_key(t, col)
    k = 2
    while k <= NPAD:
        d = k // 2
        while d >= 1:
            bit0 = (col & d) == 0
            take_min = ((col & k) == 0) == bit0
            kp = substage(kp, k, d, bit0, take_min)
            kt = substage(kt, k, d, bit0, take_min)
            d //= 2
        k *= 2

    sp_ref[...] = (kp & jnp.int32(NPAD - 1)) | ((kt & jnp.int32(NPAD - 1)) << 10)


def _run_sort(pred_pad, targ_pad):
    return pl.pallas_call(
        _sort_body,
        grid=(NSTEPS,),
        in_specs=[
            pl.BlockSpec((ROWS_BLK, NPAD), lambda i: (i, 0)),
            pl.BlockSpec((ROWS_BLK, NPAD), lambda i: (i, 0)),
        ],
        out_specs=[
            pl.BlockSpec((ROWS_BLK, NPAD), lambda i: (i, 0)),
            pl.BlockSpec((1, 1), lambda i: (0, 0)),
        ],
        out_shape=[
            jax.ShapeDtypeStruct((B, NPAD), jnp.int32),
            jax.ShapeDtypeStruct((1, 1), jnp.float32),
        ],
    )(pred_pad, targ_pad)


NC = 2
NS = 16
NW = NC * NS
RPW = B // NW
CHUNK = 16
NCHUNK = RPW // CHUNK

_VREGS = NPAD // 16


def _sc_row_compute(i, sp_v, tv_v, tr_v, ic_buf, ret_buf):
    sbase = i * NPAD
    tbase = i * N
    lanes = lax.iota(jnp.int32, 16)
    m10 = jnp.int32(NPAD - 1)
    for k in range(_VREGS):
        idx = (sp_v[pl.ds(sbase + k * 16, 16)] >> 10) & m10
        val = (lanes + (k * 16)).astype(jnp.float32)
        plsc.store_scatter(tr_v, [idx], val)
    acc = jnp.zeros((16,), jnp.float32)
    for k in range(63):
        ip = sp_v[pl.ds(sbase + k * 16, 16)] & m10
        g = plsc.load_gather(tr_v, [ip])
        p_idx = lanes + (k * 16)
        w = jnp.where(p_idx < N, p_idx, 0).astype(jnp.float32)
        acc = acc + w * g
    icn = jnp.sum(acc)
    ib = sp_v[pl.ds(sbase, 16)] & m10
    gb = plsc.load_gather(tv_v, [jnp.minimum(ib, N - 1) + tbase])
    bsum = jnp.sum(jnp.where(lanes < TOPK, gb, 0.0))
    it = sp_v[pl.ds(sbase + 992, 16)] & m10
    gt = plsc.load_gather(tv_v, [jnp.minimum(it, N - 1) + tbase])
    tmask = (lanes >= 3) & (lanes < 8)
    tsum = jnp.sum(jnp.where(tmask, gt, 0.0))
    ret = (1.0 / TOPK) * (tsum - bsum) - TCOST * 2.0

    lane0 = lanes == 0
    widx = jnp.full((16,), i, jnp.int32)
    plsc.store_scatter(ic_buf, [widx], jnp.full((16,), icn, jnp.float32),
                       mask=lane0)
    plsc.store_scatter(ret_buf, [widx], jnp.full((16,), ret, jnp.float32),
                       mask=lane0)


def _sc_body(sp_hbm, tgt_hbm, icnum_hbm, ret_hbm,
             sp_v, tv_v, tr_v, ic_buf, ret_buf):
    wid = lax.axis_index("s") * NC + lax.axis_index("c")
    base = wid * RPW

    def chunk_fn(c, carry):
        r0 = base + c * CHUNK
        pltpu.sync_copy(sp_hbm.at[pl.ds(r0 * NPAD, CHUNK * NPAD)], sp_v)
        pltpu.sync_copy(tgt_hbm.at[pl.ds(r0 * N, CHUNK * N)], tv_v)

        def row_fn(i, carry2):
            _sc_row_compute(i, sp_v, tv_v, tr_v,
                            ic_buf.at[pl.ds(c * CHUNK, CHUNK)],
                            ret_buf.at[pl.ds(c * CHUNK, CHUNK)])
            return carry2

        lax.fori_loop(0, CHUNK, row_fn, 0)
        return carry

    lax.fori_loop(0, NCHUNK, chunk_fn, 0)
    pltpu.sync_copy(ic_buf, icnum_hbm.at[pl.ds(base, RPW)])
    pltpu.sync_copy(ret_buf, ret_hbm.at[pl.ds(base, RPW)])


def _run_sc(sp_flat, tgt_flat):
    mesh = plsc.VectorSubcoreMesh(core_axis_name="c", subcore_axis_name="s")
    fn = functools.partial(
        pl.kernel,
        mesh=mesh,
        compiler_params=pltpu.CompilerParams(needs_layout_passes=False),
        out_type=[
            jax.ShapeDtypeStruct((B,), jnp.float32),
            jax.ShapeDtypeStruct((B,), jnp.float32),
        ],
        scratch_types=[
            pltpu.VMEM((CHUNK * NPAD,), jnp.int32),
            pltpu.VMEM((CHUNK * N,), jnp.float32),
            pltpu.VMEM((NPAD,), jnp.float32),
            pltpu.VMEM((RPW,), jnp.float32),
            pltpu.VMEM((RPW,), jnp.float32),
        ],
    )(_sc_body)
    return fn(sp_flat, tgt_flat)


def _combine_body(ic_ref, ret_ref, mse_ref, fw_ref, out_ref):
    icn = ic_ref[...]
    ret = ret_ref[...]
    nb = float(B)
    ic_loss = -(jnp.sum(icn) - nb * A_CONST) / (DENOM * nb)
    mean_ret = jnp.sum(ret) / nb
    var = jnp.sum((ret - mean_ret) ** 2) / (nb - 1.0)
    sharpe_loss = -(mean_ret / (jnp.sqrt(var) + 1e-8))
    l1 = jnp.sum(jnp.abs(fw_ref[...]))
    mse = mse_ref[0, 0] / (nb * float(N))
    total = mse + 0.5 * ic_loss + 0.5 * sharpe_loss + 0.0001 * l1
    out_ref[...] = jnp.reshape(total, (1, 1))


def _run_combine(icnum, ret, msesum, factor_weights):
    return pl.pallas_call(
        _combine_body,
        out_shape=jax.ShapeDtypeStruct((1, 1), jnp.float32),
    )(icnum.reshape(1, B), ret.reshape(1, B), msesum, factor_weights)


def kernel(predictions, targets, factor_weights):
    pred_pad = jnp.pad(predictions, ((0, 0), (0, NPAD - N)),
                       constant_values=jnp.inf)
    targ_pad = jnp.pad(targets, ((0, 0), (0, NPAD - N)),
                       constant_values=jnp.inf)
    spst, msesum = _run_sort(pred_pad, targ_pad)
    icnum, ret = _run_sc(spst.reshape(-1), targets.reshape(-1))
    out = _run_combine(icnum, ret, msesum, factor_weights)
    return out.reshape(())
